# baseline (speedup 1.0000x reference)
import functools

import jax
import jax.numpy as jnp
from jax import lax
from jax.experimental import pallas as pl
from jax.experimental.pallas import tpu as pltpu
from jax.experimental.pallas import tpu_sc as plsc

E = 8
H = 1024
FF = 3584
CF = 896
NF = FF // CF
T = 32
L = 16
CW = 128


def _router_sc(x, gate_w):
    mesh = plsc.VectorSubcoreMesh(
        core_axis_name="c", subcore_axis_name="s", num_cores=1
    )
    tpt = T // 16

    @functools.partial(
        pl.kernel,
        mesh=mesh,
        out_type=jax.ShapeDtypeStruct((T, CW), jnp.float32),
        scratch_types=[
            pltpu.VMEM((tpt, H), jnp.float32),
            pltpu.VMEM((E, H), jnp.float32),
            pltpu.VMEM((L,), jnp.float32),
        ],
    )
    def _body(x_hbm, gate_hbm, out_hbm, x_v, gate_v, coef_v):
        sid = lax.axis_index("s")
        pltpu.sync_copy(x_hbm.at[pl.ds(sid * tpt, tpt)], x_v)
        pltpu.sync_copy(gate_hbm, gate_v)

        lane = lax.iota(jnp.int32, L)

        def _allred(v, op):
            for sh in (8, 4, 2, 1):
                v = op(v, v.at[lane ^ sh].get(mode="promise_in_bounds"))
            return v

        for tok in range(tpt):
            accs = [jnp.zeros((L,), jnp.float32) for _ in range(E)]
            for j in range(H // L):
                xc = x_v[tok, pl.ds(j * L, L)]
                for e in range(E):
                    accs[e] = accs[e] + xc * gate_v[e, pl.ds(j * L, L)]
            lvec = jnp.full((L,), -jnp.inf, dtype=jnp.float32)
            for e in range(E):
                lvec = jnp.where(lane == e, _allred(accs[e], jnp.add), lvec)

            m1 = _allred(lvec, jnp.maximum)
            i1 = _allred(jnp.where(lvec == m1, lane, L), jnp.minimum)
            mask1 = lane == i1
            rest = jnp.where(mask1, -jnp.inf, lvec)
            m2 = _allred(rest, jnp.maximum)
            i2 = _allred(jnp.where(rest == m2, lane, L), jnp.minimum)
            top2 = mask1 | (lane == i2)
            ev = jnp.exp(lvec - m1)
            ev = jnp.where(top2, ev, 0.0)
            coef_v[...] = ev / _allred(ev, jnp.add)
            pltpu.sync_copy(
                coef_v, out_hbm.at[sid * tpt + tok, pl.ds(0, L)]
            )

    return _body(x, gate_w)


def _dot_nt(a, b):
    return lax.dot_general(
        a, b, (((1,), (1,)), ((), ())), preferred_element_type=jnp.float32
    )


def _partials_body(x_ref, w1_ref, w3_ref, w2_ref, p_ref):
    f = pl.program_id(1)

    @pl.when(f == 0)
    def _init():
        p_ref[...] = jnp.zeros_like(p_ref)

    x = x_ref[...]
    a = _dot_nt(x, w1_ref[0])
    b = _dot_nt(x, w3_ref[0])
    g = a * jax.nn.sigmoid(a) * b
    p_ref[0] += _dot_nt(g, w2_ref[0])


def _combine_body(p_ref, coef_ref, out_ref):
    cols = jax.lax.broadcasted_iota(jnp.int32, coef_ref.shape, 1)
    cvals = coef_ref[...]
    acc = jnp.zeros_like(out_ref)
    for e in range(E):
        c = jnp.sum(jnp.where(cols == e, cvals, 0.0), axis=1, keepdims=True)
        acc += c * p_ref[e]
    out_ref[...] = acc


def kernel(hidden_states, gate_w, w1, w3, w2, prefetch_expert_idx):
    b, s, h = hidden_states.shape
    t = b * s
    x = hidden_states.reshape(t, h)

    partials = pl.pallas_call(
        _partials_body,
        grid=(E, NF),
        in_specs=[
            pl.BlockSpec((t, H), lambda e, f: (0, 0)),
            pl.BlockSpec((1, CF, H), lambda e, f: (e, f, 0)),
            pl.BlockSpec((1, CF, H), lambda e, f: (e, f, 0)),
            pl.BlockSpec((1, H, CF), lambda e, f: (e, 0, f)),
        ],
        out_specs=pl.BlockSpec((1, t, H), lambda e, f: (e, 0, 0)),
        out_shape=jax.ShapeDtypeStruct((E, t, H), jnp.float32),
        compiler_params=pltpu.CompilerParams(
            dimension_semantics=("arbitrary", "arbitrary"),
        ),
    )(x, w1, w3, w2)

    coef = _router_sc(x, gate_w)

    out = pl.pallas_call(
        _combine_body,
        in_specs=[
            pl.BlockSpec((E, t, H), lambda: (0, 0, 0)),
            pl.BlockSpec((t, CW), lambda: (0, 0)),
        ],
        out_specs=pl.BlockSpec((t, H), lambda: (0, 0)),
        out_shape=jax.ShapeDtypeStruct((t, H), jnp.float32),
    )(partials, coef)

    return out.reshape(b, s, h)


# speedup vs baseline: 1.0350x; 1.0350x over previous
import functools

import jax
import jax.numpy as jnp
from jax import lax
from jax.experimental import pallas as pl
from jax.experimental.pallas import tpu as pltpu
from jax.experimental.pallas import tpu_sc as plsc

E = 8
H = 1024
FF = 3584
CF = 896
NF = FF // CF
T = 32
L = 16
CW = 128


def _router_sc(x, gate_w):
    mesh = plsc.VectorSubcoreMesh(
        core_axis_name="c", subcore_axis_name="s", num_cores=1
    )
    tpt = T // 16

    @functools.partial(
        pl.kernel,
        mesh=mesh,
        out_type=jax.ShapeDtypeStruct((T, CW), jnp.float32),
        scratch_types=[
            pltpu.VMEM((tpt, H), jnp.float32),
            pltpu.VMEM((E, H), jnp.float32),
            pltpu.VMEM((L,), jnp.float32),
        ],
    )
    def _body(x_hbm, gate_hbm, out_hbm, x_v, gate_v, coef_v):
        sid = lax.axis_index("s")
        pltpu.sync_copy(x_hbm.at[pl.ds(sid * tpt, tpt)], x_v)
        pltpu.sync_copy(gate_hbm, gate_v)

        lane = lax.iota(jnp.int32, L)

        def _allred(v, op):
            for sh in (8, 4, 2, 1):
                v = op(v, v.at[lane ^ sh].get(mode="promise_in_bounds"))
            return v

        for tok in range(tpt):
            accs = [jnp.zeros((L,), jnp.float32) for _ in range(E)]
            for j in range(H // L):
                xc = x_v[tok, pl.ds(j * L, L)]
                for e in range(E):
                    accs[e] = accs[e] + xc * gate_v[e, pl.ds(j * L, L)]
            lvec = jnp.full((L,), -jnp.inf, dtype=jnp.float32)
            for e in range(E):
                lvec = jnp.where(lane == e, _allred(accs[e], jnp.add), lvec)

            m1 = _allred(lvec, jnp.maximum)
            i1 = _allred(jnp.where(lvec == m1, lane, L), jnp.minimum)
            mask1 = lane == i1
            rest = jnp.where(mask1, -jnp.inf, lvec)
            m2 = _allred(rest, jnp.maximum)
            i2 = _allred(jnp.where(rest == m2, lane, L), jnp.minimum)
            top2 = mask1 | (lane == i2)
            ev = jnp.exp(lvec - m1)
            ev = jnp.where(top2, ev, 0.0)
            coef_v[...] = ev / _allred(ev, jnp.add)
            pltpu.sync_copy(
                coef_v, out_hbm.at[sid * tpt + tok, pl.ds(0, L)]
            )

    return _body(x, gate_w)


def _dot_nt(a, b):
    return lax.dot_general(
        a, b, (((1,), (1,)), ((), ())), preferred_element_type=jnp.float32
    )


def _partials_body(x_ref, w1_ref, w3_ref, w2_ref, p_ref):
    f = pl.program_id(1)

    @pl.when(f == 0)
    def _init():
        p_ref[...] = jnp.zeros_like(p_ref)

    x = x_ref[...]
    a = _dot_nt(x, w1_ref[0])
    b = _dot_nt(x, w3_ref[0])
    g = a * jax.nn.sigmoid(a) * b
    p_ref[0] += _dot_nt(g, w2_ref[0])


def _combine_body(p_ref, coef_ref, out_ref):
    cols = jax.lax.broadcasted_iota(jnp.int32, coef_ref.shape, 1)
    cvals = coef_ref[...]
    acc = jnp.zeros_like(out_ref)
    for e in range(E):
        c = jnp.sum(jnp.where(cols == e, cvals, 0.0), axis=1, keepdims=True)
        acc += c * p_ref[e]
    out_ref[...] = acc


def kernel(hidden_states, gate_w, w1, w3, w2, prefetch_expert_idx):
    b, s, h = hidden_states.shape
    t = b * s
    x = hidden_states.reshape(t, h)

    coef = _router_sc(x, gate_w)

    partials = pl.pallas_call(
        _partials_body,
        grid=(E, NF),
        in_specs=[
            pl.BlockSpec((t, H), lambda e, f: (0, 0)),
            pl.BlockSpec((1, CF, H), lambda e, f: (e, f, 0)),
            pl.BlockSpec((1, CF, H), lambda e, f: (e, f, 0)),
            pl.BlockSpec((1, H, CF), lambda e, f: (e, 0, f)),
        ],
        out_specs=pl.BlockSpec((1, t, H), lambda e, f: (e, 0, 0)),
        out_shape=jax.ShapeDtypeStruct((E, t, H), jnp.float32),
        compiler_params=pltpu.CompilerParams(
            dimension_semantics=("arbitrary", "arbitrary"),
        ),
    )(x, w1, w3, w2)

    out = pl.pallas_call(
        _combine_body,
        in_specs=[
            pl.BlockSpec((E, t, H), lambda: (0, 0, 0)),
            pl.BlockSpec((t, CW), lambda: (0, 0)),
        ],
        out_specs=pl.BlockSpec((t, H), lambda: (0, 0)),
        out_shape=jax.ShapeDtypeStruct((t, H), jnp.float32),
    )(partials, coef)

    return out.reshape(b, s, h)
